# baseline (speedup 1.0000x reference)
import functools

import numpy as np
import jax
import jax.numpy as jnp
from jax.experimental import pallas as pl
from jax.experimental.pallas import tpu as pltpu

CTX = 2048
TOPK = 32
NKEYS = 512
D2 = 512
ROWS_PER_BLOCK = 512
NEG_INF = float("-inf")

_pairs = [(i, j) for i in range(TOPK) for j in range(TOPK)
          if (i + 1) * (j + 1) <= TOPK]
NCAND = 128
assert len(_pairs) <= NCAND
_ci = np.array([p[0] for p in _pairs])
_cj = np.array([p[1] for p in _pairs])
_A_np = np.zeros((TOPK, NCAND), np.float32)
_B_np = np.zeros((TOPK, NCAND), np.float32)
_A_np[_ci, np.arange(len(_pairs))] = 1.0
_B_np[_cj, np.arange(len(_pairs))] = 1.0
_A512_np = _A_np * np.float32(NKEYS)
_C_np = np.full((1, NCAND), -np.inf, np.float32)
_C_np[0, : len(_pairs)] = 0.0


def _dot(a, b, precision=jax.lax.Precision.HIGHEST):
    return jax.lax.dot_general(
        a, b, (((1,), (0,)), ((), ())),
        precision=precision, preferred_element_type=jnp.float32)


def _topk_desc(v, k, payload=None, want_idx=True):
    t, n = v.shape
    lane = jax.lax.broadcasted_iota(jnp.int32, (1, n), 1)
    kiota = jax.lax.broadcasted_iota(jnp.int32, (1, k), 1)
    big = jnp.int32(np.int32(2**30))
    acc_v = jnp.full((t, k), NEG_INF, dtype=jnp.float32)
    acc_i = jnp.zeros((t, k), dtype=jnp.int32) if want_idx else None
    acc_p = None if payload is None else jnp.zeros((t, k), dtype=payload.dtype)
    for ki in range(k):
        m = jnp.max(v, axis=-1, keepdims=True)
        cand = jnp.where(v == m, lane, big)
        am = jnp.min(cand, axis=-1, keepdims=True)
        sel = cand == am
        acc_v = jnp.where(kiota == ki, m, acc_v)
        if want_idx:
            acc_i = jnp.where(kiota == ki, am, acc_i)
        if payload is not None:
            p = jnp.max(jnp.where(sel, payload, jnp.int32(-1)),
                        axis=-1, keepdims=True)
            acc_p = jnp.where(kiota == ki, p, acc_p)
        v = jnp.where(sel, NEG_INF, v)
    return acc_v, acc_i, acc_p


def _pkm_kernel(x_ref, k0_ref, k1_ref, w_ref, b_ref, a_ref, bb_ref,
                a512_ref, c_ref, out_s_ref, out_i_ref):
    xb = x_ref[...]
    w = w_ref[...]
    b = b_ref[...]
    eps = jnp.float32(1e-5)

    def ln(h):
        mu = jnp.mean(h, axis=-1, keepdims=True)
        hc = h - mu
        var = jnp.mean(hc * hc, axis=-1, keepdims=True)
        return hc / jnp.sqrt(var + eps) * w + b

    q0 = ln(xb[:, :D2])
    q1 = ln(xb[:, D2:])
    dots0 = _dot(q0, k0_ref[...], precision=jax.lax.Precision.DEFAULT)
    dots1 = _dot(q1, k1_ref[...], precision=jax.lax.Precision.DEFAULT)

    t = xb.shape[0]
    tr = 64
    sv0, si0, sv1, si1 = [], [], [], []
    for i in range(0, t, tr):
        v, idx, _ = _topk_desc(dots0[i:i + tr, :], TOPK)
        sv0.append(v)
        si0.append(idx)
        v, idx, _ = _topk_desc(dots1[i:i + tr, :], TOPK)
        sv1.append(v)
        si1.append(idx)
    sx = jnp.concatenate(sv0, axis=0)
    ix = jnp.concatenate(si0, axis=0)
    sy = jnp.concatenate(sv1, axis=0)
    iy = jnp.concatenate(si1, axis=0)

    A = a_ref[...]
    B = bb_ref[...]
    cand_s = _dot(sx, A) + _dot(sy, B) + c_ref[...]
    cand_if = _dot(ix.astype(jnp.float32), a512_ref[...]) + _dot(
        iy.astype(jnp.float32), B)
    cand_idx = cand_if.astype(jnp.int32)

    fin_s, _, fin_i = _topk_desc(cand_s, TOPK, payload=cand_idx,
                                 want_idx=False)
    out_s_ref[...] = fin_s
    out_i_ref[...] = fin_i


@jax.jit
def kernel(x, keys, norm_w, norm_b):
    rows = x.shape[0]
    t = ROWS_PER_BLOCK
    grid = (rows // t,)
    k0 = keys[:, 0, :].T
    k1 = keys[:, 1, :].T
    w2 = norm_w.reshape(1, D2)
    b2 = norm_b.reshape(1, D2)
    A = jnp.asarray(_A_np)
    B = jnp.asarray(_B_np)
    A512 = jnp.asarray(_A512_np)
    C = jnp.asarray(_C_np)

    const = lambda shape: pl.BlockSpec(shape, lambda i: (0, 0))
    out_s, out_i = pl.pallas_call(
        _pkm_kernel,
        grid=grid,
        in_specs=[
            pl.BlockSpec((t, 2 * D2), lambda i: (i, 0)),
            const((D2, NKEYS)),
            const((D2, NKEYS)),
            const((1, D2)),
            const((1, D2)),
            const((TOPK, NCAND)),
            const((TOPK, NCAND)),
            const((TOPK, NCAND)),
            const((1, NCAND)),
        ],
        out_specs=[
            pl.BlockSpec((t, TOPK), lambda i: (i, 0)),
            pl.BlockSpec((t, TOPK), lambda i: (i, 0)),
        ],
        out_shape=[
            jax.ShapeDtypeStruct((rows, TOPK), jnp.float32),
            jax.ShapeDtypeStruct((rows, TOPK), jnp.int32),
        ],
    )(x, k0, k1, w2, b2, A, B, A512, C)
    return (out_s, out_i)


# speedup vs baseline: 2.1561x; 2.1561x over previous
import functools

import numpy as np
import jax
import jax.numpy as jnp
from jax.experimental import pallas as pl
from jax.experimental.pallas import tpu as pltpu

CTX = 2048
TOPK = 32
NKEYS = 512
D2 = 512
ROWS_PER_BLOCK = 1024
NEG_INF = float("-inf")

_pairs = [(i, j) for i in range(TOPK) for j in range(TOPK)
          if (i + 1) * (j + 1) <= TOPK]
NCAND = 128
assert len(_pairs) <= NCAND
_ci = np.array([p[0] for p in _pairs])
_cj = np.array([p[1] for p in _pairs])
_A_np = np.zeros((TOPK, NCAND), np.float32)
_B_np = np.zeros((TOPK, NCAND), np.float32)
_A_np[_ci, np.arange(len(_pairs))] = 1.0
_B_np[_cj, np.arange(len(_pairs))] = 1.0
_A512_np = _A_np * np.float32(NKEYS)
_C_np = np.full((1, NCAND), -np.inf, np.float32)
_C_np[0, : len(_pairs)] = 0.0


def _dot(a, b, precision=jax.lax.Precision.HIGHEST):
    return jax.lax.dot_general(
        a, b, (((1,), (0,)), ((), ())),
        precision=precision, preferred_element_type=jnp.float32)


def _topk_desc(v, k, payload=None, want_idx=True):
    t, n = v.shape
    lane = jax.lax.broadcasted_iota(jnp.int32, (1, n), 1)
    kiota = jax.lax.broadcasted_iota(jnp.int32, (1, k), 1)
    big = jnp.int32(np.int32(2**30))
    acc_v = jnp.full((t, k), NEG_INF, dtype=jnp.float32)
    acc_i = jnp.zeros((t, k), dtype=jnp.int32) if want_idx else None
    acc_p = None if payload is None else jnp.zeros((t, k), dtype=payload.dtype)
    for ki in range(k):
        m = jnp.max(v, axis=-1, keepdims=True)
        cand = jnp.where(v == m, lane, big)
        am = jnp.min(cand, axis=-1, keepdims=True)
        sel = cand == am
        acc_v = jnp.where(kiota == ki, m, acc_v)
        if want_idx:
            acc_i = jnp.where(kiota == ki, am, acc_i)
        if payload is not None:
            p = jnp.max(jnp.where(sel, payload, jnp.int32(-1)),
                        axis=-1, keepdims=True)
            acc_p = jnp.where(kiota == ki, p, acc_p)
        v = jnp.where(sel, NEG_INF, v)
    return acc_v, acc_i, acc_p


def _pkm_kernel(x_ref, k0_ref, k1_ref, w_ref, b_ref, a_ref, bb_ref,
                a512_ref, c_ref, out_s_ref, out_i_ref):
    xb = x_ref[...]
    w = w_ref[...]
    b = b_ref[...]
    eps = jnp.float32(1e-5)

    def ln(h):
        mu = jnp.mean(h, axis=-1, keepdims=True)
        hc = h - mu
        var = jnp.mean(hc * hc, axis=-1, keepdims=True)
        return hc / jnp.sqrt(var + eps) * w + b

    q0 = ln(xb[:, :D2])
    q1 = ln(xb[:, D2:])
    dots0 = _dot(q0, k0_ref[...], precision=jax.lax.Precision.DEFAULT)
    dots1 = _dot(q1, k1_ref[...], precision=jax.lax.Precision.DEFAULT)

    t = xb.shape[0]
    d_all = jnp.concatenate([dots0, dots1], axis=0)
    s_all, i_all, _ = _topk_desc(d_all, TOPK)
    sx, ix = s_all[:t], i_all[:t]
    sy, iy = s_all[t:], i_all[t:]

    A = a_ref[...]
    B = bb_ref[...]
    cand_s = _dot(sx, A) + _dot(sy, B) + c_ref[...]
    cand_if = _dot(ix.astype(jnp.float32), a512_ref[...]) + _dot(
        iy.astype(jnp.float32), B)
    cand_idx = cand_if.astype(jnp.int32)

    fin_s, _, fin_i = _topk_desc(cand_s, TOPK, payload=cand_idx,
                                 want_idx=False)
    out_s_ref[...] = fin_s
    out_i_ref[...] = fin_i


@jax.jit
def kernel(x, keys, norm_w, norm_b):
    rows = x.shape[0]
    t = ROWS_PER_BLOCK
    grid = (rows // t,)
    k0 = keys[:, 0, :].T
    k1 = keys[:, 1, :].T
    w2 = norm_w.reshape(1, D2)
    b2 = norm_b.reshape(1, D2)
    A = jnp.asarray(_A_np)
    B = jnp.asarray(_B_np)
    A512 = jnp.asarray(_A512_np)
    C = jnp.asarray(_C_np)

    const = lambda shape: pl.BlockSpec(shape, lambda i: (0, 0))
    out_s, out_i = pl.pallas_call(
        _pkm_kernel,
        grid=grid,
        in_specs=[
            pl.BlockSpec((t, 2 * D2), lambda i: (i, 0)),
            const((D2, NKEYS)),
            const((D2, NKEYS)),
            const((1, D2)),
            const((1, D2)),
            const((TOPK, NCAND)),
            const((TOPK, NCAND)),
            const((TOPK, NCAND)),
            const((1, NCAND)),
        ],
        out_specs=[
            pl.BlockSpec((t, TOPK), lambda i: (i, 0)),
            pl.BlockSpec((t, TOPK), lambda i: (i, 0)),
        ],
        out_shape=[
            jax.ShapeDtypeStruct((rows, TOPK), jnp.float32),
            jax.ShapeDtypeStruct((rows, TOPK), jnp.int32),
        ],
    )(x, k0, k1, w2, b2, A, B, A512, C)
    return (out_s, out_i)
